# Initial kernel scaffold; baseline (speedup 1.0000x reference)
import jax
import jax.numpy as jnp
from jax.experimental import pallas as pl


def kernel(pretensor, finetensor):
    raise NotImplementedError("write your pallas kernel here")


# speedup vs baseline: 69.2401x; 69.2401x over previous
import functools

import jax
import jax.numpy as jnp
from jax import lax
from jax.experimental import pallas as pl
from jax.experimental.pallas import tpu as pltpu
from jax.experimental.pallas import tpu_sc as plsc

R = 4096
N = R * R
K = int(0.1 * N)
SIGMOID_BIAS = 5.0

NC, NS, L = 2, 16, 16
NW = NC * NS
ROWS_W = R // NW
CR, CC = 8, 1024
CHUNKS_R = ROWS_W // CR
CHUNKS_C = R // CC
NCHUNK = CHUNKS_R * CHUNKS_C
VPR = CC // L

B1, B2, B3 = 1024, 2048, 1024
U1, U2, U3 = 2, 2, 2

_mesh = plsc.VectorSubcoreMesh(core_axis_name="c", subcore_axis_name="s")
_sc_params = pltpu.CompilerParams(
    needs_layout_passes=False, use_tc_tiling_on_sc=True)


def _zero(ref, nwords):
    z = jnp.zeros((L,), jnp.int32)

    def body(i, _):
        ref[pl.ds(i * L, L)] = z
        return 0

    lax.fori_loop(0, nwords // L, body, 0)


def _extract0(sel_v):
    lanes = lax.iota(jnp.int32, L)
    v = sel_v[pl.ds(0, L)]
    return jnp.sum(jnp.where(lanes == 0, v, 0))


def _wid():
    return lax.axis_index("s") * NC + lax.axis_index("c")


def _chunk_window(wid, c, cc=CC, chunks_c=CHUNKS_C, cr=CR):
    r0 = wid * ROWS_W + (c // chunks_c) * cr
    c0 = (c % chunks_c) * cc
    return r0, c0


def _hist_reduce_out(hist_v, out_v, out_hbm, wid, nbins, nu):
    def red_body(j, _):
        acc = hist_v[pl.ds(j * L, L)]
        for cc in range(1, nu * L):
            acc = acc + hist_v[pl.ds(cc * nbins + j * L, L)]
        out_v[pl.ds(j * L, L)] = acc
        return 0

    lax.fori_loop(0, nbins // L, red_body, 0)
    pltpu.sync_copy(out_v, out_hbm.at[pl.ds(wid * nbins, nbins)])


def _scatter_rows(buf_fn, hist_v, nbins, nu, bin_fn, vpr, cr):
    lanes = lax.iota(jnp.int32, L)
    ones = jnp.ones((L,), jnp.int32)
    cvecs = [(u * L + lanes) * nbins for u in range(nu)]
    for r in range(cr):
        @plsc.parallel_loop(0, vpr // nu, unroll=2)
        def _(i0):
            for u in range(nu):
                i = i0 * nu + u
                bits = buf_fn(r, i)
                bin_idx, m = bin_fn(bits)
                idx = cvecs[u] | bin_idx
                plsc.addupdate_scatter(hist_v, [idx], ones, mask=m)


@functools.partial(
    pl.kernel,
    out_type=(
        jax.ShapeDtypeStruct((R, R), jnp.int32),
        jax.ShapeDtypeStruct((NW * B1,), jnp.int32),
    ),
    mesh=_mesh,
    compiler_params=_sc_params,
    scratch_types=[
        pltpu.VMEM((CR, CC), jnp.float32),
        pltpu.VMEM((CR, CC), jnp.float32),
        pltpu.VMEM((CR, CC), jnp.float32),
        pltpu.VMEM((CR, CC), jnp.float32),
        pltpu.VMEM((CR, CC), jnp.int32),
        pltpu.VMEM((CR, CC), jnp.int32),
        pltpu.SemaphoreType.DMA,
        pltpu.SemaphoreType.DMA,
        pltpu.SemaphoreType.DMA,
        pltpu.SemaphoreType.DMA,
        pltpu.SemaphoreType.DMA,
        pltpu.SemaphoreType.DMA,
        pltpu.VMEM((U1 * L * B1,), jnp.int32),
        pltpu.VMEM((B1,), jnp.int32),
    ],
)
def _pass1(pre_hbm, fine_hbm, bits_hbm, out_hbm,
           p0, p1, f0, f1, o0, o1, sp0, sp1, sf0, sf1, so0, so1,
           hist_v, out_v):
    wid = _wid()
    pres = (p0, p1)
    fins = (f0, f1)
    outs = (o0, o1)
    psem = (sp0, sp1)
    fsem = (sf0, sf1)
    osem = (so0, so1)
    _zero(hist_v, U1 * L * B1)

    def start_in(c, b):
        r0, c0 = _chunk_window(wid, c)
        src = lambda h: h.at[pl.ds(r0, CR), pl.ds(c0, CC)]
        pltpu.async_copy(src(pre_hbm), pres[b], psem[b])
        pltpu.async_copy(src(fine_hbm), fins[b], fsem[b])

    def wait_in(b):
        pltpu.make_async_copy(
            pre_hbm.at[pl.ds(0, CR), pl.ds(0, CC)], pres[b], psem[b]).wait()
        pltpu.make_async_copy(
            fine_hbm.at[pl.ds(0, CR), pl.ds(0, CC)], fins[b], fsem[b]).wait()

    def start_out(c, b):
        r0, c0 = _chunk_window(wid, c)
        pltpu.async_copy(outs[b], bits_hbm.at[pl.ds(r0, CR), pl.ds(c0, CC)],
                         osem[b])

    def wait_out(b):
        pltpu.make_async_copy(
            outs[b], bits_hbm.at[pl.ds(0, CR), pl.ds(0, CC)], osem[b]).wait()

    lanes = lax.iota(jnp.int32, L)
    ones = jnp.ones((L,), jnp.int32)

    def compute(c, b):
        pre_v, fine_v, out_v2 = pres[b], fins[b], outs[b]
        for r in range(CR):
            @plsc.parallel_loop(0, VPR, unroll=U1)
            def _(i):
                p = pre_v[r, pl.ds(i * L, L)]
                f = fine_v[r, pl.ds(i * L, L)]
                sbits = lax.bitcast_convert_type(f - p, jnp.int32)
                out_v2[r, pl.ds(i * L, L)] = sbits
                abits = sbits & 0x7FFFFFFF
                bin_idx = lax.shift_right_logical(abits, 21)
                cbase = (i & (U1 - 1)) * L + lanes
                idx = cbase * B1 + bin_idx
                plsc.addupdate_scatter(hist_v, [idx], ones, mask=None)
        start_out(c, b)

    start_in(0, 0)

    def chunk_body(ch, _):
        c0 = ch * 2

        @pl.when(c0 + 1 < NCHUNK)
        def _():
            start_in(c0 + 1, 1)

        wait_in(0)

        @pl.when(c0 >= 2)
        def _():
            wait_out(0)

        compute(c0, 0)

        @pl.when(c0 + 2 < NCHUNK)
        def _():
            start_in(c0 + 2, 0)

        @pl.when(c0 + 1 < NCHUNK)
        def _():
            wait_in(1)

            @pl.when(c0 >= 1)
            def _():
                wait_out(1)

            compute(c0 + 1, 1)

        return 0

    lax.fori_loop(0, (NCHUNK + 1) // 2, chunk_body, 0)
    wait_out(0)
    wait_out(1)
    _hist_reduce_out(hist_v, out_v, out_hbm, wid, B1, U1)


_CC2 = 2048
_CHUNKS_C2 = R // _CC2
_VPR2 = _CC2 // L


def _bits_pass(nbins, nu, cr2):
    nchunk2 = (ROWS_W // cr2) * _CHUNKS_C2

    def make(bin_fn_from_sel):
        @functools.partial(
            pl.kernel,
            out_type=jax.ShapeDtypeStruct((NW * nbins,), jnp.int32),
            mesh=_mesh,
            compiler_params=_sc_params,
            scratch_types=[
                pltpu.VMEM((cr2, _CC2), jnp.int32),
                pltpu.VMEM((cr2, _CC2), jnp.int32),
                pltpu.SemaphoreType.DMA,
                pltpu.SemaphoreType.DMA,
                pltpu.VMEM((nu * L * nbins,), jnp.int32),
                pltpu.VMEM((nbins,), jnp.int32),
                pltpu.VMEM((L,), jnp.int32),
            ],
        )
        def k(bits_hbm, sel_hbm, out_hbm,
              b0, b1, s0, s1, hist_v, out_v, sel_v):
            wid = _wid()
            bufs = (b0, b1)
            sems = (s0, s1)
            pltpu.sync_copy(sel_hbm.at[0], sel_v)
            bin_fn = bin_fn_from_sel(_extract0(sel_v))
            _zero(hist_v, nu * L * nbins)

            def start(c, b):
                r0, c0 = _chunk_window(wid, c, _CC2, _CHUNKS_C2, cr2)
                pltpu.async_copy(
                    bits_hbm.at[pl.ds(r0, cr2), pl.ds(c0, _CC2)], bufs[b],
                    sems[b])

            def wait(b):
                pltpu.make_async_copy(
                    bits_hbm.at[pl.ds(0, cr2), pl.ds(0, _CC2)], bufs[b],
                    sems[b]).wait()

            def compute(b):
                buf = bufs[b]
                _scatter_rows(
                    lambda r, i: buf[r, pl.ds(i * L, L)],
                    hist_v, nbins, nu, bin_fn, _VPR2, cr2)

            start(0, 0)

            def chunk_body(ch, _):
                c0 = ch * 2

                @pl.when(c0 + 1 < nchunk2)
                def _():
                    start(c0 + 1, 1)

                wait(0)
                compute(0)

                @pl.when(c0 + 2 < nchunk2)
                def _():
                    start(c0 + 2, 0)

                @pl.when(c0 + 1 < nchunk2)
                def _():
                    wait(1)
                    compute(1)

                return 0

            lax.fori_loop(0, (nchunk2 + 1) // 2, chunk_body, 0)
            _hist_reduce_out(hist_v, out_v, out_hbm, wid, nbins, nu)

        return k

    return make


def _p2_bins(b1):
    def bin_fn(sbits):
        m = (lax.shift_right_logical(sbits, 21) & 0x3FF) == b1
        bin2 = lax.shift_right_logical(sbits, 10) & 0x7FF
        return bin2, m
    return bin_fn


def _p3_bins(key):
    def bin_fn(sbits):
        m = (lax.shift_right_logical(sbits, 10) & 0x1FFFFF) == key
        bin3 = sbits & 0x3FF
        return bin3, m
    return bin_fn


_pass2 = _bits_pass(B2, U2, 8)(_p2_bins)
_pass3 = _bits_pass(B3, U3, 8)(_p3_bins)


def _sel(h_ref, nb, target):
    h = jnp.sum(h_ref[...].astype(jnp.float32), axis=0, keepdims=True)
    ii = lax.broadcasted_iota(jnp.int32, (nb, nb), 0)
    jj = lax.broadcasted_iota(jnp.int32, (nb, nb), 1)
    tri = (ii >= jj).astype(jnp.float32)
    s = jnp.dot(h, tri, preferred_element_type=jnp.float32)
    b = jnp.sum((s >= target).astype(jnp.float32)) - 1.0
    ids = lax.broadcasted_iota(jnp.int32, (1, nb), 1).astype(jnp.float32)
    hb = jnp.sum(jnp.where(ids > b, h, 0.0))
    return b, hb


def _sel_out(ref, a, b):
    ids = lax.broadcasted_iota(jnp.int32, (1, L), 1)
    ref[...] = jnp.where(ids == 0, a, jnp.where(ids == 1, b, 0))


def _sel1_body(h1_ref, out_ref):
    b1, s1 = _sel(h1_ref, B1, float(K))
    _sel_out(out_ref, b1.astype(jnp.int32), s1.astype(jnp.int32))


_sel1 = pl.pallas_call(
    _sel1_body,
    out_shape=jax.ShapeDtypeStruct((1, L), jnp.int32),
)


def _sel2_body(h2_ref, sel1_ref, out_ref):
    b1 = sel1_ref[0, 0]
    r1 = float(K) - sel1_ref[0, 1].astype(jnp.float32)
    b2, s2 = _sel(h2_ref, B2, r1)
    key = lax.shift_left(b1, 11) | b2.astype(jnp.int32)
    _sel_out(out_ref, key, s2.astype(jnp.int32))


_sel2 = pl.pallas_call(
    _sel2_body,
    out_shape=jax.ShapeDtypeStruct((1, L), jnp.int32),
)


_BLK_R = 512


def _mask_body(h3_ref, sel1_ref, sel2_ref, bits_ref,
               mask_ref, delta_ref, prop_ref, thr_s):
    @pl.when(pl.program_id(0) == 0)
    def _():
        s1 = sel1_ref[0, 1].astype(jnp.float32)
        key = sel2_ref[0, 0]
        s2 = sel2_ref[0, 1].astype(jnp.float32)
        r2 = float(K) - s1 - s2
        b3, s3 = _sel(h3_ref, B3, r2)
        thr = lax.shift_left(key, 10) | b3.astype(jnp.int32)
        thr_s[0] = thr
        prop_ref[...] = ((s1 + s2 + s3) / float(N)).reshape(1, 1)

    t = thr_s[0]
    sbits = bits_ref[...]
    tv = lax.bitcast_convert_type(sbits, jnp.float32)
    cond = (sbits & 0x7FFFFFFF) > t
    m = jnp.where(cond, SIGMOID_BIAS, -SIGMOID_BIAS).astype(jnp.float32)
    mask_ref[...] = m
    delta_ref[...] = (1.0 / (1.0 + jnp.exp(-m))) * tv


_mask_kernel = pl.pallas_call(
    _mask_body,
    grid=(R // _BLK_R,),
    in_specs=[
        pl.BlockSpec((NW, B3), lambda i: (0, 0)),
        pl.BlockSpec((1, L), lambda i: (0, 0)),
        pl.BlockSpec((1, L), lambda i: (0, 0)),
        pl.BlockSpec((_BLK_R, R), lambda i: (i, 0)),
    ],
    out_specs=[
        pl.BlockSpec((_BLK_R, R), lambda i: (i, 0)),
        pl.BlockSpec((_BLK_R, R), lambda i: (i, 0)),
        pl.BlockSpec((1, 1), lambda i: (0, 0)),
    ],
    out_shape=(
        jax.ShapeDtypeStruct((R, R), jnp.float32),
        jax.ShapeDtypeStruct((R, R), jnp.float32),
        jax.ShapeDtypeStruct((1, 1), jnp.float32),
    ),
    scratch_shapes=[pltpu.SMEM((1,), jnp.int32)],
    compiler_params=pltpu.CompilerParams(
        dimension_semantics=("arbitrary",)),
)


def kernel(pretensor, finetensor):
    bits, h1 = _pass1(pretensor, finetensor)
    sel1 = _sel1(h1.reshape(NW, B1))
    h2 = _pass2(bits, sel1)
    sel2 = _sel2(h2.reshape(NW, B2), sel1)
    h3 = _pass3(bits, sel2)
    mask, delta, prop = _mask_kernel(h3.reshape(NW, B3), sel1, sel2, bits)
    return mask, delta, prop.reshape(())
